# baseline (speedup 1.0000x reference)
import functools

import jax
import jax.numpy as jnp
from jax import lax
from jax.experimental import pallas as pl
from jax.experimental.pallas import tpu as pltpu
from jax.experimental.pallas import tpu_sc as plsc

F = 512
H = 4
DH = F // H
N0, N1, N2 = 8192, 4096, 1024
NC, NS = 2, 16
NW = NC * NS


def _sc_gather_l1(features, sn_cat, raw_idx):
  mesh = plsc.VectorSubcoreMesh(core_axis_name="c", subcore_axis_name="s")

  @functools.partial(
      pl.kernel,
      out_type=jax.ShapeDtypeStruct((24576, F), jnp.float32),
      mesh=mesh,
      compiler_params=pltpu.CompilerParams(needs_layout_passes=False),
      scratch_types=[
          pltpu.VMEM((6, 128), jnp.int32),
          pltpu.VMEM((2 * N0,), jnp.int32),
          pltpu.VMEM((6, 128), jnp.int32),
          pltpu.VMEM((128, F), jnp.float32),
          pltpu.SemaphoreType.DMA,
      ],
  )
  def k(feat_hbm, sn_hbm, idx_hbm, out_hbm, idxraw_v, sn_v, comp_v, rows_v,
        sem):
    wid = lax.axis_index("s") * NC + lax.axis_index("c")
    pltpu.sync_copy(idx_hbm.at[wid], idxraw_v)
    pltpu.sync_copy(sn_hbm, sn_v)
    for c in range(6):
      for i in range(8):
        idx16 = idxraw_v[c, pl.ds(i * 16, 16)]
        comp_v[c, pl.ds(i * 16, 16)] = plsc.load_gather(sn_v, [idx16])
    for c in range(6):
      pltpu.async_copy(feat_hbm.at[comp_v.at[c]], rows_v, sem).wait()
      pltpu.sync_copy(rows_v, out_hbm.at[pl.ds(wid * 768 + c * 128, 128)])

  return k(features, sn_cat, raw_idx)


def _sc_gather_l2(x1_cat, raw_idx):
  mesh = plsc.VectorSubcoreMesh(core_axis_name="c", subcore_axis_name="s")

  @functools.partial(
      pl.kernel,
      out_type=jax.ShapeDtypeStruct((10240, F), jnp.float32),
      mesh=mesh,
      scratch_types=[
          pltpu.VMEM((5, 64), jnp.int32),
          pltpu.VMEM((64, F), jnp.float32),
          pltpu.SemaphoreType.DMA,
      ],
  )
  def k(x_hbm, idx_hbm, out_hbm, idx_v, rows_v, sem):
    wid = lax.axis_index("s") * NC + lax.axis_index("c")
    pltpu.sync_copy(idx_hbm.at[wid], idx_v)
    for c in range(5):
      pltpu.async_copy(x_hbm.at[idx_v.at[c]], rows_v, sem).wait()
      pltpu.sync_copy(rows_v, out_hbm.at[pl.ds(wid * 320 + c * 64, 64)])

  return k(x1_cat, raw_idx)


def _layer_body(dif_ref, src_ref, dst_ref, wa_ref, wb_ref, a_ref, c_ref,
                out_ref, acc_ref, *, bm, rel):
  k = pl.program_id(1)

  @pl.when(k == 0)
  def _():
    acc_ref[...] = jnp.zeros_like(acc_ref)

  acc_ref[...] += jnp.dot(dif_ref[...],
                          src_ref[...].astype(jnp.bfloat16),
                          preferred_element_type=jnp.float32)

  @pl.when(k == pl.num_programs(1) - 1)
  def _():
    h = jnp.dot(acc_ref[...], wa_ref[...], preferred_element_type=jnp.float32)
    h += jnp.dot(dst_ref[...], wb_ref[...], preferred_element_type=jnp.float32)
    h = jnp.maximum(h, 0.0)
    scores = jnp.dot(h, a_ref[...], preferred_element_type=jnp.float32)
    scores += c_ref[rel:rel + 1, :]
    w = 1.0 / (1.0 + jnp.exp(-scores))
    hh = h.reshape(bm, H, DH)
    out_ref[...] = (hh * w[:, :, None]).reshape(bm, F)


def _tc_layer(dif_mat, rows, w_mat, a_mat, c_vec, rel, src_off, dst_off,
              bm=512, bk=512):
  m, kk = dif_mat.shape
  grid = (m // bm, kk // bk)
  so, do = src_off // bk, dst_off // bm
  return pl.pallas_call(
      functools.partial(_layer_body, bm=bm, rel=rel),
      grid=grid,
      in_specs=[
          pl.BlockSpec((bm, bk), lambda i, j: (i, j)),
          pl.BlockSpec((bk, F), lambda i, j, so=so: (so + j, 0)),
          pl.BlockSpec((bm, F), lambda i, j, do=do: (do + i, 0)),
          pl.BlockSpec((F, F), lambda i, j: (0, 0)),
          pl.BlockSpec((F, F), lambda i, j: (1, 0)),
          pl.BlockSpec((F, H), lambda i, j: (0, 0)),
          pl.BlockSpec((2, H), lambda i, j: (0, 0)),
      ],
      out_specs=pl.BlockSpec((bm, F), lambda i, j: (i, 0)),
      out_shape=jax.ShapeDtypeStruct((m, F), jnp.float32),
      scratch_shapes=[pltpu.VMEM((bm, F), jnp.float32)],
  )(dif_mat, rows, rows, w_mat, w_mat, a_mat, c_vec)


def _final_body(x0_ref, x1_ref, wc_ref, out_ref):
  s = x0_ref[...] + x1_ref[...]
  n = s * lax.rsqrt(jnp.maximum(jnp.sum(s * s, axis=1, keepdims=True), 1e-12))
  logits = jnp.dot(n, wc_ref[...], preferred_element_type=jnp.float32)
  m = jnp.max(logits, axis=1, keepdims=True)
  e = jnp.exp(logits - m)
  out_ref[...] = e / jnp.sum(e, axis=1, keepdims=True)


def _tc_final(x2_0, x2_1, wc):
  n_cls = wc.shape[1]
  return pl.pallas_call(
      _final_body,
      out_shape=jax.ShapeDtypeStruct((N2, n_cls), jnp.float32),
  )(x2_0, x2_1, wc)


def kernel(features, src_nodes_0, dstsrc2src_0_1, dstsrc2dst_0_1, dif_mat_0_1,
           dstsrc2src_0_2, dstsrc2dst_0_2, dif_mat_0_2, src_nodes_1,
           dstsrc2src_1_1, dstsrc2dst_1_1, dif_mat_1_1, dstsrc2src_1_2,
           dstsrc2dst_1_2, dif_mat_1_2, W1, W2, attention_vec,
           relation_vectors, Wc):
  a1 = attention_vec[:F, 0]
  a2 = attention_vec[F:, 0].reshape(H, DH)
  heads = jnp.arange(F, dtype=jnp.int32) // DH
  a_mat = a1[:, None] * (heads[:, None] == jnp.arange(H)[None, :])
  c_vec = jnp.sum(relation_vectors.reshape(2, H, DH) * a2[None], axis=2)

  idx1 = jnp.concatenate([dstsrc2src_0_1, dstsrc2dst_0_1,
                          dstsrc2src_1_1 + N0,
                          dstsrc2dst_1_1 + N0]).reshape(32, 6, 128)
  sn_cat = jnp.concatenate([src_nodes_0, src_nodes_1])
  rows1 = _sc_gather_l1(features, sn_cat, idx1)

  bf = jnp.bfloat16
  x1_0 = _tc_layer(dif_mat_0_1.astype(bf), rows1, W1, a_mat, c_vec,
                   0, 0, 8192)
  x1_1 = _tc_layer(dif_mat_1_1.astype(bf), rows1, W1, a_mat, c_vec,
                   1, 12288, 20480)

  idx2 = jnp.concatenate([dstsrc2src_0_2, dstsrc2dst_0_2,
                          dstsrc2src_1_2 + N1,
                          dstsrc2dst_1_2 + N1]).reshape(32, 5, 64)
  rows2 = _sc_gather_l2(jnp.concatenate([x1_0, x1_1]), idx2)

  x2_0 = _tc_layer(dif_mat_0_2.astype(bf), rows2, W2, a_mat, c_vec,
                   0, 0, 4096)
  x2_1 = _tc_layer(dif_mat_1_2.astype(bf), rows2, W2, a_mat, c_vec,
                   1, 5120, 9216)

  return _tc_final(x2_0, x2_1, Wc)


# speedup vs baseline: 1.3541x; 1.3541x over previous
import functools

import jax
import jax.numpy as jnp
from jax import lax
from jax.experimental import pallas as pl
from jax.experimental.pallas import tpu as pltpu
from jax.experimental.pallas import tpu_sc as plsc

F = 512
H = 4
DH = F // H
N0, N1, N2 = 8192, 4096, 1024
NC, NS = 2, 16
NW = NC * NS


def _sc_gather_l1(features, sn_cat, raw_idx):
  mesh = plsc.VectorSubcoreMesh(core_axis_name="c", subcore_axis_name="s")

  @functools.partial(
      pl.kernel,
      out_type=jax.ShapeDtypeStruct((24576, F), jnp.float32),
      mesh=mesh,
      compiler_params=pltpu.CompilerParams(needs_layout_passes=False),
      scratch_types=[
          pltpu.VMEM((6, 128), jnp.int32),
          pltpu.VMEM((2 * N0,), jnp.int32),
          pltpu.VMEM((6, 128), jnp.int32),
          pltpu.VMEM((128, F), jnp.float32),
          pltpu.SemaphoreType.DMA,
      ],
  )
  def k(feat_hbm, sn_hbm, idx_hbm, out_hbm, idxraw_v, sn_v, comp_v, rows_v,
        sem):
    wid = lax.axis_index("s") * NC + lax.axis_index("c")
    pltpu.sync_copy(idx_hbm.at[wid], idxraw_v)
    pltpu.sync_copy(sn_hbm, sn_v)
    for c in range(6):
      for i in range(8):
        idx16 = idxraw_v[c, pl.ds(i * 16, 16)]
        comp_v[c, pl.ds(i * 16, 16)] = plsc.load_gather(sn_v, [idx16])
    for c in range(6):
      pltpu.async_copy(feat_hbm.at[comp_v.at[c]], rows_v, sem).wait()
      pltpu.sync_copy(rows_v, out_hbm.at[pl.ds(wid * 768 + c * 128, 128)])

  return k(features, sn_cat, raw_idx)


def _sc_gather_l2(x1_cat, raw_idx):
  mesh = plsc.VectorSubcoreMesh(core_axis_name="c", subcore_axis_name="s")

  @functools.partial(
      pl.kernel,
      out_type=jax.ShapeDtypeStruct((10240, F), jnp.float32),
      mesh=mesh,
      scratch_types=[
          pltpu.VMEM((5, 64), jnp.int32),
          pltpu.VMEM((64, F), jnp.float32),
          pltpu.SemaphoreType.DMA,
      ],
  )
  def k(x_hbm, idx_hbm, out_hbm, idx_v, rows_v, sem):
    wid = lax.axis_index("s") * NC + lax.axis_index("c")
    pltpu.sync_copy(idx_hbm.at[wid], idx_v)
    for c in range(5):
      pltpu.async_copy(x_hbm.at[idx_v.at[c]], rows_v, sem).wait()
      pltpu.sync_copy(rows_v, out_hbm.at[pl.ds(wid * 320 + c * 64, 64)])

  return k(x1_cat, raw_idx)


def _layer_body(dif_ref, src_ref, dst_ref, wa_ref, wb_ref, a_ref, c_ref,
                out_ref, acc_ref, *, bm, bk, rel):
  k = pl.program_id(1)

  @pl.when(k == 0)
  def _():
    acc_ref[...] = jnp.zeros_like(acc_ref)

  acc_ref[...] += jnp.dot(dif_ref[...],
                          src_ref[pl.ds(k * bk, bk), :],
                          preferred_element_type=jnp.float32)

  @pl.when(k == pl.num_programs(1) - 1)
  def _():
    h = jnp.dot(acc_ref[...], wa_ref[...], preferred_element_type=jnp.float32)
    h += jnp.dot(dst_ref[...], wb_ref[...], preferred_element_type=jnp.float32)
    h = jnp.maximum(h, 0.0)
    scores = jnp.dot(h, a_ref[...], preferred_element_type=jnp.float32)
    scores += c_ref[rel:rel + 1, :]
    w = 1.0 / (1.0 + jnp.exp(-scores))
    hh = h.reshape(bm, H, DH)
    out_ref[...] = (hh * w[:, :, None]).reshape(bm, F)


def _tc_layer(dif_mat, rows, w_mat, a_mat, c_vec, rel, src_off, dst_off,
              bm=512, bk=512):
  m, kk = dif_mat.shape
  grid = (m // bm, kk // bk)
  so, do = src_off // kk, dst_off // bm
  return pl.pallas_call(
      functools.partial(_layer_body, bm=bm, bk=bk, rel=rel),
      grid=grid,
      in_specs=[
          pl.BlockSpec((bm, bk), lambda i, j: (i, j)),
          pl.BlockSpec((kk, F), lambda i, j, so=so: (so, 0)),
          pl.BlockSpec((bm, F), lambda i, j, do=do: (do + i, 0)),
          pl.BlockSpec((F, F), lambda i, j: (0, 0)),
          pl.BlockSpec((F, F), lambda i, j: (1, 0)),
          pl.BlockSpec((F, H), lambda i, j: (0, 0)),
          pl.BlockSpec((2, H), lambda i, j: (0, 0)),
      ],
      out_specs=pl.BlockSpec((bm, F), lambda i, j: (i, 0)),
      out_shape=jax.ShapeDtypeStruct((m, F), jnp.float32),
      scratch_shapes=[pltpu.VMEM((bm, F), jnp.float32)],
  )(dif_mat, rows, rows, w_mat, w_mat, a_mat, c_vec)


def _final_body(x0_ref, x1_ref, wc_ref, out_ref):
  s = x0_ref[...] + x1_ref[...]
  n = s * lax.rsqrt(jnp.maximum(jnp.sum(s * s, axis=1, keepdims=True), 1e-12))
  logits = jnp.dot(n, wc_ref[...], preferred_element_type=jnp.float32)
  m = jnp.max(logits, axis=1, keepdims=True)
  e = jnp.exp(logits - m)
  out_ref[...] = e / jnp.sum(e, axis=1, keepdims=True)


def _tc_final(x2_0, x2_1, wc):
  n_cls = wc.shape[1]
  return pl.pallas_call(
      _final_body,
      out_shape=jax.ShapeDtypeStruct((N2, n_cls), jnp.float32),
  )(x2_0, x2_1, wc)


def kernel(features, src_nodes_0, dstsrc2src_0_1, dstsrc2dst_0_1, dif_mat_0_1,
           dstsrc2src_0_2, dstsrc2dst_0_2, dif_mat_0_2, src_nodes_1,
           dstsrc2src_1_1, dstsrc2dst_1_1, dif_mat_1_1, dstsrc2src_1_2,
           dstsrc2dst_1_2, dif_mat_1_2, W1, W2, attention_vec,
           relation_vectors, Wc):
  a1 = attention_vec[:F, 0]
  a2 = attention_vec[F:, 0].reshape(H, DH)
  heads = jnp.arange(F, dtype=jnp.int32) // DH
  a_mat = a1[:, None] * (heads[:, None] == jnp.arange(H)[None, :])
  c_vec = jnp.sum(relation_vectors.reshape(2, H, DH) * a2[None], axis=2)

  idx1 = jnp.concatenate([dstsrc2src_0_1, dstsrc2src_1_1 + N0,
                          dstsrc2dst_0_1,
                          dstsrc2dst_1_1 + N0]).reshape(32, 6, 128)
  sn_cat = jnp.concatenate([src_nodes_0, src_nodes_1])
  rows1 = _sc_gather_l1(features, sn_cat, idx1)

  x1_0 = _tc_layer(dif_mat_0_1, rows1, W1, a_mat, c_vec, 0, 0, 16384)
  x1_1 = _tc_layer(dif_mat_1_1, rows1, W1, a_mat, c_vec, 1, 8192, 20480)

  idx2 = jnp.concatenate([dstsrc2src_0_2, dstsrc2src_1_2 + N1,
                          dstsrc2dst_0_2,
                          dstsrc2dst_1_2 + N1]).reshape(32, 5, 64)
  rows2 = _sc_gather_l2(jnp.concatenate([x1_0, x1_1]), idx2)

  x2_0 = _tc_layer(dif_mat_0_2, rows2, W2, a_mat, c_vec, 0, 0, 8192)
  x2_1 = _tc_layer(dif_mat_1_2, rows2, W2, a_mat, c_vec, 1, 4096, 9216)

  return _tc_final(x2_0, x2_1, Wc)


# speedup vs baseline: 1.3549x; 1.0005x over previous
import functools

import jax
import jax.numpy as jnp
from jax import lax
from jax.experimental import pallas as pl
from jax.experimental.pallas import tpu as pltpu
from jax.experimental.pallas import tpu_sc as plsc

F = 512
H = 4
DH = F // H
N0, N1, N2 = 8192, 4096, 1024
NC, NS = 2, 16
NW = NC * NS


def _sc_gather_l1(features, sn_cat, raw_idx):
  mesh = plsc.VectorSubcoreMesh(core_axis_name="c", subcore_axis_name="s")

  @functools.partial(
      pl.kernel,
      out_type=jax.ShapeDtypeStruct((24576, F), jnp.float32),
      mesh=mesh,
      compiler_params=pltpu.CompilerParams(needs_layout_passes=False),
      scratch_types=[
          pltpu.VMEM((6, 128), jnp.int32),
          pltpu.VMEM((2 * N0,), jnp.int32),
          pltpu.VMEM((6, 128), jnp.int32),
          pltpu.VMEM((128, F), jnp.float32),
          pltpu.SemaphoreType.DMA,
      ],
  )
  def k(feat_hbm, sn_hbm, idx_hbm, out_hbm, idxraw_v, sn_v, comp_v, rows_v,
        sem):
    wid = lax.axis_index("s") * NC + lax.axis_index("c")
    pltpu.sync_copy(idx_hbm.at[wid], idxraw_v)
    pltpu.sync_copy(sn_hbm, sn_v)
    for c in range(6):
      for i in range(8):
        idx16 = idxraw_v[c, pl.ds(i * 16, 16)]
        comp_v[c, pl.ds(i * 16, 16)] = plsc.load_gather(sn_v, [idx16])
    for c in range(6):
      pltpu.async_copy(feat_hbm.at[comp_v.at[c]], rows_v, sem).wait()
      pltpu.sync_copy(rows_v, out_hbm.at[pl.ds(wid * 768 + c * 128, 128)])

  return k(features, sn_cat, raw_idx)


def _sc_gather_l2(x1_cat, raw_idx):
  mesh = plsc.VectorSubcoreMesh(core_axis_name="c", subcore_axis_name="s")

  @functools.partial(
      pl.kernel,
      out_type=jax.ShapeDtypeStruct((10240, F), jnp.float32),
      mesh=mesh,
      scratch_types=[
          pltpu.VMEM((5, 64), jnp.int32),
          pltpu.VMEM((64, F), jnp.float32),
          pltpu.SemaphoreType.DMA,
      ],
  )
  def k(x_hbm, idx_hbm, out_hbm, idx_v, rows_v, sem):
    wid = lax.axis_index("s") * NC + lax.axis_index("c")
    pltpu.sync_copy(idx_hbm.at[wid], idx_v)
    for c in range(5):
      pltpu.async_copy(x_hbm.at[idx_v.at[c]], rows_v, sem).wait()
      pltpu.sync_copy(rows_v, out_hbm.at[pl.ds(wid * 320 + c * 64, 64)])

  return k(x1_cat, raw_idx)


def _layer_body(dif_ref, src_ref, dst_ref, wa_ref, wb_ref, a_ref, c_ref,
                out_ref, acc_ref, *, bm, bk, rel):
  k = pl.program_id(1)

  @pl.when(k == 0)
  def _():
    acc_ref[...] = jnp.zeros_like(acc_ref)

  acc_ref[...] += jnp.dot(dif_ref[...].astype(jnp.bfloat16),
                          src_ref[pl.ds(k * bk, bk), :].astype(jnp.bfloat16),
                          preferred_element_type=jnp.float32)

  @pl.when(k == pl.num_programs(1) - 1)
  def _():
    h = jnp.dot(acc_ref[...], wa_ref[...], preferred_element_type=jnp.float32)
    h += jnp.dot(dst_ref[...], wb_ref[...], preferred_element_type=jnp.float32)
    h = jnp.maximum(h, 0.0)
    scores = jnp.dot(h, a_ref[...], preferred_element_type=jnp.float32)
    scores += c_ref[rel:rel + 1, :]
    w = 1.0 / (1.0 + jnp.exp(-scores))
    hh = h.reshape(bm, H, DH)
    out_ref[...] = (hh * w[:, :, None]).reshape(bm, F)


def _tc_layer(dif_mat, rows, w_mat, a_mat, c_vec, rel, src_off, dst_off,
              bm=512, bk=512):
  m, kk = dif_mat.shape
  grid = (m // bm, kk // bk)
  so, do = src_off // kk, dst_off // bm
  return pl.pallas_call(
      functools.partial(_layer_body, bm=bm, bk=bk, rel=rel),
      grid=grid,
      in_specs=[
          pl.BlockSpec((bm, bk), lambda i, j: (i, j)),
          pl.BlockSpec((kk, F), lambda i, j, so=so: (so, 0)),
          pl.BlockSpec((bm, F), lambda i, j, do=do: (do + i, 0)),
          pl.BlockSpec((F, F), lambda i, j: (0, 0)),
          pl.BlockSpec((F, F), lambda i, j: (1, 0)),
          pl.BlockSpec((F, H), lambda i, j: (0, 0)),
          pl.BlockSpec((2, H), lambda i, j: (0, 0)),
      ],
      out_specs=pl.BlockSpec((bm, F), lambda i, j: (i, 0)),
      out_shape=jax.ShapeDtypeStruct((m, F), jnp.float32),
      scratch_shapes=[pltpu.VMEM((bm, F), jnp.float32)],
  )(dif_mat, rows, rows, w_mat, w_mat, a_mat, c_vec)


def _final_body(x0_ref, x1_ref, wc_ref, out_ref):
  s = x0_ref[...] + x1_ref[...]
  n = s * lax.rsqrt(jnp.maximum(jnp.sum(s * s, axis=1, keepdims=True), 1e-12))
  logits = jnp.dot(n, wc_ref[...], preferred_element_type=jnp.float32)
  m = jnp.max(logits, axis=1, keepdims=True)
  e = jnp.exp(logits - m)
  out_ref[...] = e / jnp.sum(e, axis=1, keepdims=True)


def _tc_final(x2_0, x2_1, wc):
  n_cls = wc.shape[1]
  return pl.pallas_call(
      _final_body,
      out_shape=jax.ShapeDtypeStruct((N2, n_cls), jnp.float32),
  )(x2_0, x2_1, wc)


def kernel(features, src_nodes_0, dstsrc2src_0_1, dstsrc2dst_0_1, dif_mat_0_1,
           dstsrc2src_0_2, dstsrc2dst_0_2, dif_mat_0_2, src_nodes_1,
           dstsrc2src_1_1, dstsrc2dst_1_1, dif_mat_1_1, dstsrc2src_1_2,
           dstsrc2dst_1_2, dif_mat_1_2, W1, W2, attention_vec,
           relation_vectors, Wc):
  a1 = attention_vec[:F, 0]
  a2 = attention_vec[F:, 0].reshape(H, DH)
  heads = jnp.arange(F, dtype=jnp.int32) // DH
  a_mat = a1[:, None] * (heads[:, None] == jnp.arange(H)[None, :])
  c_vec = jnp.sum(relation_vectors.reshape(2, H, DH) * a2[None], axis=2)

  idx1 = jnp.concatenate([dstsrc2src_0_1, dstsrc2src_1_1 + N0,
                          dstsrc2dst_0_1,
                          dstsrc2dst_1_1 + N0]).reshape(32, 6, 128)
  sn_cat = jnp.concatenate([src_nodes_0, src_nodes_1])
  rows1 = _sc_gather_l1(features, sn_cat, idx1)

  x1_0 = _tc_layer(dif_mat_0_1, rows1, W1, a_mat, c_vec, 0, 0, 16384)
  x1_1 = _tc_layer(dif_mat_1_1, rows1, W1, a_mat, c_vec, 1, 8192, 20480)

  idx2 = jnp.concatenate([dstsrc2src_0_2, dstsrc2src_1_2 + N1,
                          dstsrc2dst_0_2,
                          dstsrc2dst_1_2 + N1]).reshape(32, 5, 64)
  rows2 = _sc_gather_l2(jnp.concatenate([x1_0, x1_1]), idx2)

  x2_0 = _tc_layer(dif_mat_0_2, rows2, W2, a_mat, c_vec, 0, 0, 8192)
  x2_1 = _tc_layer(dif_mat_1_2, rows2, W2, a_mat, c_vec, 1, 4096, 9216)

  return _tc_final(x2_0, x2_1, Wc)


# speedup vs baseline: 1.8944x; 1.3982x over previous
import functools

import jax
import jax.numpy as jnp
from jax import lax
from jax.experimental import pallas as pl
from jax.experimental.pallas import tpu as pltpu
from jax.experimental.pallas import tpu_sc as plsc

F = 512
H = 4
DH = F // H
N0, N1, N2 = 8192, 4096, 1024
NC, NS = 2, 16
NW = NC * NS


def _sc_gather_l1(features, sn_cat, raw_idx):
  mesh = plsc.VectorSubcoreMesh(core_axis_name="c", subcore_axis_name="s")

  @functools.partial(
      pl.kernel,
      out_type=jax.ShapeDtypeStruct((24576, F), jnp.float32),
      mesh=mesh,
      compiler_params=pltpu.CompilerParams(needs_layout_passes=False),
      scratch_types=[
          pltpu.VMEM((6, 128), jnp.int32),
          pltpu.VMEM((2 * N0,), jnp.int32),
          pltpu.VMEM((6, 128), jnp.int32),
          pltpu.VMEM((128, F), jnp.float32),
          pltpu.SemaphoreType.DMA,
      ],
  )
  def k(feat_hbm, sn_hbm, idx_hbm, out_hbm, idxraw_v, sn_v, comp_v, rows_v,
        sem):
    wid = lax.axis_index("s") * NC + lax.axis_index("c")
    pltpu.sync_copy(idx_hbm.at[wid], idxraw_v)
    pltpu.sync_copy(sn_hbm, sn_v)
    for c in range(6):
      for i in range(8):
        idx16 = idxraw_v[c, pl.ds(i * 16, 16)]
        comp_v[c, pl.ds(i * 16, 16)] = plsc.load_gather(sn_v, [idx16])
    for c in range(6):
      pltpu.async_copy(feat_hbm.at[comp_v.at[c]], rows_v, sem).wait()
      pltpu.sync_copy(rows_v, out_hbm.at[pl.ds(wid * 768 + c * 128, 128)])

  return k(features, sn_cat, raw_idx)


def _sc_gather_l2(x1_cat, raw_idx):
  mesh = plsc.VectorSubcoreMesh(core_axis_name="c", subcore_axis_name="s")

  @functools.partial(
      pl.kernel,
      out_type=jax.ShapeDtypeStruct((10240, F), jnp.float32),
      mesh=mesh,
      scratch_types=[
          pltpu.VMEM((5, 64), jnp.int32),
          pltpu.VMEM((64, F), jnp.float32),
          pltpu.SemaphoreType.DMA,
      ],
  )
  def k(x_hbm, idx_hbm, out_hbm, idx_v, rows_v, sem):
    wid = lax.axis_index("s") * NC + lax.axis_index("c")
    pltpu.sync_copy(idx_hbm.at[wid], idx_v)
    for c in range(5):
      pltpu.async_copy(x_hbm.at[idx_v.at[c]], rows_v, sem).wait()
      pltpu.sync_copy(rows_v, out_hbm.at[pl.ds(wid * 320 + c * 64, 64)])

  return k(x1_cat, raw_idx)


def _layer_body(dif_ref, src_ref, dst_ref, wa_ref, wb_ref, a_ref, c_ref,
                out_ref, acc_ref, *, bm, bk, rel):
  k = pl.program_id(1)

  @pl.when(k == 0)
  def _():
    acc_ref[...] = jnp.zeros_like(acc_ref)

  acc_ref[...] += jnp.dot(dif_ref[...],
                          src_ref[pl.ds(k * bk, bk), :],
                          preferred_element_type=jnp.float32)

  @pl.when(k == pl.num_programs(1) - 1)
  def _():
    h = jnp.dot(acc_ref[...], wa_ref[...], preferred_element_type=jnp.float32)
    h += jnp.dot(dst_ref[...], wb_ref[...], preferred_element_type=jnp.float32)
    h = jnp.maximum(h, 0.0)
    scores = jnp.dot(h, a_ref[...], preferred_element_type=jnp.float32)
    scores += c_ref[rel:rel + 1, :]
    w = 1.0 / (1.0 + jnp.exp(-scores))
    hh = h.reshape(bm, H, DH)
    out_ref[...] = (hh * w[:, :, None]).reshape(bm, F)


def _tc_layer(dif_mat, rows, w_mat, a_mat, c_vec, rel, src_off, dst_off,
              bm=512, bk=2048):
  m, kk = dif_mat.shape
  grid = (m // bm, kk // bk)
  so, do = src_off // kk, dst_off // bm
  return pl.pallas_call(
      functools.partial(_layer_body, bm=bm, bk=bk, rel=rel),
      grid=grid,
      in_specs=[
          pl.BlockSpec((bm, bk), lambda i, j: (i, j)),
          pl.BlockSpec((kk, F), lambda i, j, so=so: (so, 0)),
          pl.BlockSpec((bm, F), lambda i, j, do=do: (do + i, 0)),
          pl.BlockSpec((F, F), lambda i, j: (0, 0)),
          pl.BlockSpec((F, F), lambda i, j: (1, 0)),
          pl.BlockSpec((F, H), lambda i, j: (0, 0)),
          pl.BlockSpec((2, H), lambda i, j: (0, 0)),
      ],
      out_specs=pl.BlockSpec((bm, F), lambda i, j: (i, 0)),
      out_shape=jax.ShapeDtypeStruct((m, F), jnp.float32),
      scratch_shapes=[pltpu.VMEM((bm, F), jnp.float32)],
  )(dif_mat, rows, rows, w_mat, w_mat, a_mat, c_vec)


def _final_body(x0_ref, x1_ref, wc_ref, out_ref):
  s = x0_ref[...] + x1_ref[...]
  n = s * lax.rsqrt(jnp.maximum(jnp.sum(s * s, axis=1, keepdims=True), 1e-12))
  logits = jnp.dot(n, wc_ref[...], preferred_element_type=jnp.float32)
  m = jnp.max(logits, axis=1, keepdims=True)
  e = jnp.exp(logits - m)
  out_ref[...] = e / jnp.sum(e, axis=1, keepdims=True)


def _tc_final(x2_0, x2_1, wc):
  n_cls = wc.shape[1]
  return pl.pallas_call(
      _final_body,
      out_shape=jax.ShapeDtypeStruct((N2, n_cls), jnp.float32),
  )(x2_0, x2_1, wc)


def kernel(features, src_nodes_0, dstsrc2src_0_1, dstsrc2dst_0_1, dif_mat_0_1,
           dstsrc2src_0_2, dstsrc2dst_0_2, dif_mat_0_2, src_nodes_1,
           dstsrc2src_1_1, dstsrc2dst_1_1, dif_mat_1_1, dstsrc2src_1_2,
           dstsrc2dst_1_2, dif_mat_1_2, W1, W2, attention_vec,
           relation_vectors, Wc):
  a1 = attention_vec[:F, 0]
  a2 = attention_vec[F:, 0].reshape(H, DH)
  heads = jnp.arange(F, dtype=jnp.int32) // DH
  a_mat = a1[:, None] * (heads[:, None] == jnp.arange(H)[None, :])
  c_vec = jnp.sum(relation_vectors.reshape(2, H, DH) * a2[None], axis=2)

  idx1 = jnp.concatenate([dstsrc2src_0_1, dstsrc2src_1_1 + N0,
                          dstsrc2dst_0_1,
                          dstsrc2dst_1_1 + N0]).reshape(32, 6, 128)
  sn_cat = jnp.concatenate([src_nodes_0, src_nodes_1])
  rows1 = _sc_gather_l1(features, sn_cat, idx1)

  x1_0 = _tc_layer(dif_mat_0_1, rows1, W1, a_mat, c_vec, 0, 0, 16384)
  x1_1 = _tc_layer(dif_mat_1_1, rows1, W1, a_mat, c_vec, 1, 8192, 20480)

  idx2 = jnp.concatenate([dstsrc2src_0_2, dstsrc2src_1_2 + N1,
                          dstsrc2dst_0_2,
                          dstsrc2dst_1_2 + N1]).reshape(32, 5, 64)
  rows2 = _sc_gather_l2(jnp.concatenate([x1_0, x1_1]), idx2)

  x2_0 = _tc_layer(dif_mat_0_2, rows2, W2, a_mat, c_vec, 0, 0, 8192)
  x2_1 = _tc_layer(dif_mat_1_2, rows2, W2, a_mat, c_vec, 1, 4096, 9216)

  return _tc_final(x2_0, x2_1, Wc)


# speedup vs baseline: 2.0025x; 1.0570x over previous
import functools

import jax
import jax.numpy as jnp
from jax import lax
from jax.experimental import pallas as pl
from jax.experimental.pallas import tpu as pltpu
from jax.experimental.pallas import tpu_sc as plsc

F = 512
H = 4
DH = F // H
N0, N1, N2 = 8192, 4096, 1024
NC, NS = 2, 16
NW = NC * NS


def _sc_gather_l1(features, sn_cat, raw_idx):
  mesh = plsc.VectorSubcoreMesh(core_axis_name="c", subcore_axis_name="s")

  @functools.partial(
      pl.kernel,
      out_type=jax.ShapeDtypeStruct((24576, F), jnp.float32),
      mesh=mesh,
      compiler_params=pltpu.CompilerParams(needs_layout_passes=False),
      scratch_types=[
          pltpu.VMEM((6, 128), jnp.int32),
          pltpu.VMEM((2 * N0,), jnp.int32),
          pltpu.VMEM((6, 128), jnp.int32),
          pltpu.VMEM((128, F), jnp.float32),
          pltpu.SemaphoreType.DMA,
      ],
  )
  def k(feat_hbm, sn_hbm, idx_hbm, out_hbm, idxraw_v, sn_v, comp_v, rows_v,
        sem):
    wid = lax.axis_index("s") * NC + lax.axis_index("c")
    pltpu.sync_copy(idx_hbm.at[wid], idxraw_v)
    pltpu.sync_copy(sn_hbm, sn_v)
    for c in range(6):
      for i in range(8):
        idx16 = idxraw_v[c, pl.ds(i * 16, 16)]
        comp_v[c, pl.ds(i * 16, 16)] = plsc.load_gather(sn_v, [idx16])
    for c in range(6):
      pltpu.async_copy(feat_hbm.at[comp_v.at[c]], rows_v, sem).wait()
      pltpu.sync_copy(rows_v, out_hbm.at[pl.ds(wid * 768 + c * 128, 128)])

  return k(features, sn_cat, raw_idx)


def _sc_gather_l2(x1_cat, raw_idx):
  mesh = plsc.VectorSubcoreMesh(core_axis_name="c", subcore_axis_name="s")

  @functools.partial(
      pl.kernel,
      out_type=jax.ShapeDtypeStruct((10240, F), jnp.float32),
      mesh=mesh,
      scratch_types=[
          pltpu.VMEM((5, 64), jnp.int32),
          pltpu.VMEM((64, F), jnp.float32),
          pltpu.SemaphoreType.DMA,
      ],
  )
  def k(x_hbm, idx_hbm, out_hbm, idx_v, rows_v, sem):
    wid = lax.axis_index("s") * NC + lax.axis_index("c")
    pltpu.sync_copy(idx_hbm.at[wid], idx_v)
    for c in range(5):
      pltpu.async_copy(x_hbm.at[idx_v.at[c]], rows_v, sem).wait()
      pltpu.sync_copy(rows_v, out_hbm.at[pl.ds(wid * 320 + c * 64, 64)])

  return k(x1_cat, raw_idx)


def _layer_body(dif_ref, src_ref, dst_ref, wa_ref, wb_ref, a_ref, c_ref,
                out_ref, acc_ref, *, bm, bk, rel):
  k = pl.program_id(1)

  @pl.when(k == 0)
  def _():
    acc_ref[...] = jnp.zeros_like(acc_ref)

  acc_ref[...] += jnp.dot(dif_ref[...],
                          src_ref[pl.ds(k * bk, bk), :],
                          preferred_element_type=jnp.float32)

  @pl.when(k == pl.num_programs(1) - 1)
  def _():
    h = jnp.dot(acc_ref[...], wa_ref[...], preferred_element_type=jnp.float32)
    h += jnp.dot(dst_ref[...], wb_ref[...], preferred_element_type=jnp.float32)
    h = jnp.maximum(h, 0.0)
    scores = jnp.dot(h, a_ref[...], preferred_element_type=jnp.float32)
    scores += c_ref[rel:rel + 1, :]
    w = 1.0 / (1.0 + jnp.exp(-scores))
    hh = h.reshape(bm, H, DH)
    out_ref[...] = (hh * w[:, :, None]).reshape(bm, F)


def _tc_layer(dif_mat, rows, w_mat, a_mat, c_vec, rel, src_off, dst_off,
              bm=512, bk=4096):
  m, kk = dif_mat.shape
  grid = (m // bm, kk // bk)
  so, do = src_off // kk, dst_off // bm
  return pl.pallas_call(
      functools.partial(_layer_body, bm=bm, bk=bk, rel=rel),
      grid=grid,
      in_specs=[
          pl.BlockSpec((bm, bk), lambda i, j: (i, j)),
          pl.BlockSpec((kk, F), lambda i, j, so=so: (so, 0)),
          pl.BlockSpec((bm, F), lambda i, j, do=do: (do + i, 0)),
          pl.BlockSpec((F, F), lambda i, j: (0, 0)),
          pl.BlockSpec((F, F), lambda i, j: (1, 0)),
          pl.BlockSpec((F, H), lambda i, j: (0, 0)),
          pl.BlockSpec((2, H), lambda i, j: (0, 0)),
      ],
      out_specs=pl.BlockSpec((bm, F), lambda i, j: (i, 0)),
      out_shape=jax.ShapeDtypeStruct((m, F), jnp.float32),
      scratch_shapes=[pltpu.VMEM((bm, F), jnp.float32)],
  )(dif_mat, rows, rows, w_mat, w_mat, a_mat, c_vec)


def _final_body(x0_ref, x1_ref, wc_ref, out_ref):
  s = x0_ref[...] + x1_ref[...]
  n = s * lax.rsqrt(jnp.maximum(jnp.sum(s * s, axis=1, keepdims=True), 1e-12))
  logits = jnp.dot(n, wc_ref[...], preferred_element_type=jnp.float32)
  m = jnp.max(logits, axis=1, keepdims=True)
  e = jnp.exp(logits - m)
  out_ref[...] = e / jnp.sum(e, axis=1, keepdims=True)


def _tc_final(x2_0, x2_1, wc):
  n_cls = wc.shape[1]
  return pl.pallas_call(
      _final_body,
      out_shape=jax.ShapeDtypeStruct((N2, n_cls), jnp.float32),
  )(x2_0, x2_1, wc)


def kernel(features, src_nodes_0, dstsrc2src_0_1, dstsrc2dst_0_1, dif_mat_0_1,
           dstsrc2src_0_2, dstsrc2dst_0_2, dif_mat_0_2, src_nodes_1,
           dstsrc2src_1_1, dstsrc2dst_1_1, dif_mat_1_1, dstsrc2src_1_2,
           dstsrc2dst_1_2, dif_mat_1_2, W1, W2, attention_vec,
           relation_vectors, Wc):
  a1 = attention_vec[:F, 0]
  a2 = attention_vec[F:, 0].reshape(H, DH)
  heads = jnp.arange(F, dtype=jnp.int32) // DH
  a_mat = a1[:, None] * (heads[:, None] == jnp.arange(H)[None, :])
  c_vec = jnp.sum(relation_vectors.reshape(2, H, DH) * a2[None], axis=2)

  idx1 = jnp.concatenate([dstsrc2src_0_1, dstsrc2src_1_1 + N0,
                          dstsrc2dst_0_1,
                          dstsrc2dst_1_1 + N0]).reshape(32, 6, 128)
  sn_cat = jnp.concatenate([src_nodes_0, src_nodes_1])
  rows1 = _sc_gather_l1(features, sn_cat, idx1)

  x1_0 = _tc_layer(dif_mat_0_1, rows1, W1, a_mat, c_vec, 0, 0, 16384)
  x1_1 = _tc_layer(dif_mat_1_1, rows1, W1, a_mat, c_vec, 1, 8192, 20480)

  idx2 = jnp.concatenate([dstsrc2src_0_2, dstsrc2src_1_2 + N1,
                          dstsrc2dst_0_2,
                          dstsrc2dst_1_2 + N1]).reshape(32, 5, 64)
  rows2 = _sc_gather_l2(jnp.concatenate([x1_0, x1_1]), idx2)

  x2_0 = _tc_layer(dif_mat_0_2, rows2, W2, a_mat, c_vec, 0, 0, 8192)
  x2_1 = _tc_layer(dif_mat_1_2, rows2, W2, a_mat, c_vec, 1, 4096, 9216)

  return _tc_final(x2_0, x2_1, Wc)


# speedup vs baseline: 2.0918x; 1.0446x over previous
import functools

import jax
import jax.numpy as jnp
from jax import lax
from jax.experimental import pallas as pl
from jax.experimental.pallas import tpu as pltpu
from jax.experimental.pallas import tpu_sc as plsc

F = 512
H = 4
DH = F // H
N0, N1, N2 = 8192, 4096, 1024
NC, NS = 2, 16
NW = NC * NS


def _sc_gather_l1(features, sn_cat, raw_idx):
  mesh = plsc.VectorSubcoreMesh(core_axis_name="c", subcore_axis_name="s")

  @functools.partial(
      pl.kernel,
      out_type=jax.ShapeDtypeStruct((24576, F), jnp.float32),
      mesh=mesh,
      compiler_params=pltpu.CompilerParams(needs_layout_passes=False),
      scratch_types=[
          pltpu.VMEM((6, 128), jnp.int32),
          pltpu.VMEM((2 * N0,), jnp.int32),
          pltpu.VMEM((6, 128), jnp.int32),
          pltpu.VMEM((128, F), jnp.float32),
          pltpu.SemaphoreType.DMA,
      ],
  )
  def k(feat_hbm, sn_hbm, idx_hbm, out_hbm, idxraw_v, sn_v, comp_v, rows_v,
        sem):
    wid = lax.axis_index("s") * NC + lax.axis_index("c")
    pltpu.sync_copy(idx_hbm.at[wid], idxraw_v)
    pltpu.sync_copy(sn_hbm, sn_v)
    for c in range(6):
      for i in range(8):
        idx16 = idxraw_v[c, pl.ds(i * 16, 16)]
        comp_v[c, pl.ds(i * 16, 16)] = plsc.load_gather(sn_v, [idx16])
    for c in range(6):
      pltpu.async_copy(feat_hbm.at[comp_v.at[c]], rows_v, sem).wait()
      pltpu.sync_copy(rows_v, out_hbm.at[pl.ds(wid * 768 + c * 128, 128)])

  return k(features, sn_cat, raw_idx)


def _sc_gather_l2(x1_cat, raw_idx):
  mesh = plsc.VectorSubcoreMesh(core_axis_name="c", subcore_axis_name="s")

  @functools.partial(
      pl.kernel,
      out_type=jax.ShapeDtypeStruct((10240, F), jnp.float32),
      mesh=mesh,
      scratch_types=[
          pltpu.VMEM((5, 64), jnp.int32),
          pltpu.VMEM((64, F), jnp.float32),
          pltpu.SemaphoreType.DMA,
      ],
  )
  def k(x_hbm, idx_hbm, out_hbm, idx_v, rows_v, sem):
    wid = lax.axis_index("s") * NC + lax.axis_index("c")
    pltpu.sync_copy(idx_hbm.at[wid], idx_v)
    for c in range(5):
      pltpu.async_copy(x_hbm.at[idx_v.at[c]], rows_v, sem).wait()
      pltpu.sync_copy(rows_v, out_hbm.at[pl.ds(wid * 320 + c * 64, 64)])

  return k(x1_cat, raw_idx)


def _layer_body(dif_ref, src_ref, dst_ref, wa_ref, wb_ref, a_ref, c_ref,
                out_ref, acc_ref, *, bm, bk, rel):
  k = pl.program_id(1)

  @pl.when(k == 0)
  def _():
    acc_ref[...] = jnp.zeros_like(acc_ref)

  acc_ref[...] += jnp.dot(dif_ref[...],
                          src_ref[pl.ds(k * bk, bk), :],
                          preferred_element_type=jnp.float32)

  @pl.when(k == pl.num_programs(1) - 1)
  def _():
    h = jnp.dot(acc_ref[...], wa_ref[...], preferred_element_type=jnp.float32)
    h += jnp.dot(dst_ref[...], wb_ref[...], preferred_element_type=jnp.float32)
    h = jnp.maximum(h, 0.0)
    scores = jnp.dot(h, a_ref[...], preferred_element_type=jnp.float32)
    scores += c_ref[rel:rel + 1, :]
    w = 1.0 / (1.0 + jnp.exp(-scores))
    hh = h.reshape(bm, H, DH)
    out_ref[...] = (hh * w[:, :, None]).reshape(bm, F)


def _tc_layer(dif_mat, rows, w_mat, a_mat, c_vec, rel, src_off, dst_off,
              bm=512, bk=8192):
  m, kk = dif_mat.shape
  bk = min(bk, kk)
  grid = (m // bm, kk // bk)
  so, do = src_off // kk, dst_off // bm
  return pl.pallas_call(
      functools.partial(_layer_body, bm=bm, bk=bk, rel=rel),
      grid=grid,
      in_specs=[
          pl.BlockSpec((bm, bk), lambda i, j: (i, j)),
          pl.BlockSpec((kk, F), lambda i, j, so=so: (so, 0)),
          pl.BlockSpec((bm, F), lambda i, j, do=do: (do + i, 0)),
          pl.BlockSpec((F, F), lambda i, j: (0, 0)),
          pl.BlockSpec((F, F), lambda i, j: (1, 0)),
          pl.BlockSpec((F, H), lambda i, j: (0, 0)),
          pl.BlockSpec((2, H), lambda i, j: (0, 0)),
      ],
      out_specs=pl.BlockSpec((bm, F), lambda i, j: (i, 0)),
      out_shape=jax.ShapeDtypeStruct((m, F), jnp.float32),
      scratch_shapes=[pltpu.VMEM((bm, F), jnp.float32)],
  )(dif_mat, rows, rows, w_mat, w_mat, a_mat, c_vec)


def _final_body(x0_ref, x1_ref, wc_ref, out_ref):
  s = x0_ref[...] + x1_ref[...]
  n = s * lax.rsqrt(jnp.maximum(jnp.sum(s * s, axis=1, keepdims=True), 1e-12))
  logits = jnp.dot(n, wc_ref[...], preferred_element_type=jnp.float32)
  m = jnp.max(logits, axis=1, keepdims=True)
  e = jnp.exp(logits - m)
  out_ref[...] = e / jnp.sum(e, axis=1, keepdims=True)


def _tc_final(x2_0, x2_1, wc):
  n_cls = wc.shape[1]
  return pl.pallas_call(
      _final_body,
      out_shape=jax.ShapeDtypeStruct((N2, n_cls), jnp.float32),
  )(x2_0, x2_1, wc)


def kernel(features, src_nodes_0, dstsrc2src_0_1, dstsrc2dst_0_1, dif_mat_0_1,
           dstsrc2src_0_2, dstsrc2dst_0_2, dif_mat_0_2, src_nodes_1,
           dstsrc2src_1_1, dstsrc2dst_1_1, dif_mat_1_1, dstsrc2src_1_2,
           dstsrc2dst_1_2, dif_mat_1_2, W1, W2, attention_vec,
           relation_vectors, Wc):
  a1 = attention_vec[:F, 0]
  a2 = attention_vec[F:, 0].reshape(H, DH)
  heads = jnp.arange(F, dtype=jnp.int32) // DH
  a_mat = a1[:, None] * (heads[:, None] == jnp.arange(H)[None, :])
  c_vec = jnp.sum(relation_vectors.reshape(2, H, DH) * a2[None], axis=2)

  idx1 = jnp.concatenate([dstsrc2src_0_1, dstsrc2src_1_1 + N0,
                          dstsrc2dst_0_1,
                          dstsrc2dst_1_1 + N0]).reshape(32, 6, 128)
  sn_cat = jnp.concatenate([src_nodes_0, src_nodes_1])
  rows1 = _sc_gather_l1(features, sn_cat, idx1)

  x1_0 = _tc_layer(dif_mat_0_1, rows1, W1, a_mat, c_vec, 0, 0, 16384)
  x1_1 = _tc_layer(dif_mat_1_1, rows1, W1, a_mat, c_vec, 1, 8192, 20480)

  idx2 = jnp.concatenate([dstsrc2src_0_2, dstsrc2src_1_2 + N1,
                          dstsrc2dst_0_2,
                          dstsrc2dst_1_2 + N1]).reshape(32, 5, 64)
  rows2 = _sc_gather_l2(jnp.concatenate([x1_0, x1_1]), idx2)

  x2_0 = _tc_layer(dif_mat_0_2, rows2, W2, a_mat, c_vec, 0, 0, 8192)
  x2_1 = _tc_layer(dif_mat_1_2, rows2, W2, a_mat, c_vec, 1, 4096, 9216)

  return _tc_final(x2_0, x2_1, Wc)


# speedup vs baseline: 2.0925x; 1.0003x over previous
import functools

import jax
import jax.numpy as jnp
from jax import lax
from jax.experimental import pallas as pl
from jax.experimental.pallas import tpu as pltpu
from jax.experimental.pallas import tpu_sc as plsc

F = 512
H = 4
DH = F // H
N0, N1, N2 = 8192, 4096, 1024
NC, NS = 2, 16
NW = NC * NS


def _sc_gather_l1(features, sn_cat, raw_idx):
  mesh = plsc.VectorSubcoreMesh(core_axis_name="c", subcore_axis_name="s")

  @functools.partial(
      pl.kernel,
      out_type=jax.ShapeDtypeStruct((24576, F), jnp.float32),
      mesh=mesh,
      compiler_params=pltpu.CompilerParams(needs_layout_passes=False),
      scratch_types=[
          pltpu.VMEM((6, 128), jnp.int32),
          pltpu.VMEM((2 * N0,), jnp.int32),
          pltpu.VMEM((6, 128), jnp.int32),
          pltpu.VMEM((128, F), jnp.float32),
          pltpu.SemaphoreType.DMA,
      ],
  )
  def k(feat_hbm, sn_hbm, idx_hbm, out_hbm, idxraw_v, sn_v, comp_v, rows_v,
        sem):
    wid = lax.axis_index("s") * NC + lax.axis_index("c")
    pltpu.sync_copy(idx_hbm.at[wid], idxraw_v)
    pltpu.sync_copy(sn_hbm, sn_v)
    for c in range(6):
      for i in range(8):
        idx16 = idxraw_v[c, pl.ds(i * 16, 16)]
        comp_v[c, pl.ds(i * 16, 16)] = plsc.load_gather(sn_v, [idx16])
    for c in range(6):
      pltpu.async_copy(feat_hbm.at[comp_v.at[c]], rows_v, sem).wait()
      pltpu.sync_copy(rows_v, out_hbm.at[pl.ds(wid * 768 + c * 128, 128)])

  return k(features, sn_cat, raw_idx)


def _sc_gather_l2(x1_cat, raw_idx):
  mesh = plsc.VectorSubcoreMesh(core_axis_name="c", subcore_axis_name="s")

  @functools.partial(
      pl.kernel,
      out_type=jax.ShapeDtypeStruct((10240, F), jnp.float32),
      mesh=mesh,
      scratch_types=[
          pltpu.VMEM((5, 64), jnp.int32),
          pltpu.VMEM((64, F), jnp.float32),
          pltpu.SemaphoreType.DMA,
      ],
  )
  def k(x_hbm, idx_hbm, out_hbm, idx_v, rows_v, sem):
    wid = lax.axis_index("s") * NC + lax.axis_index("c")
    pltpu.sync_copy(idx_hbm.at[wid], idx_v)
    for c in range(5):
      pltpu.async_copy(x_hbm.at[idx_v.at[c]], rows_v, sem).wait()
      pltpu.sync_copy(rows_v, out_hbm.at[pl.ds(wid * 320 + c * 64, 64)])

  return k(x1_cat, raw_idx)


def _layer_body(dif_ref, src_ref, dst_ref, wa_ref, wb_ref, a_ref, c_ref,
                out_ref, *, bm, rel):
  agg = jnp.dot(dif_ref[...], src_ref[...], preferred_element_type=jnp.float32)
  h = jnp.dot(agg, wa_ref[...], preferred_element_type=jnp.float32)
  h += jnp.dot(dst_ref[...], wb_ref[...], preferred_element_type=jnp.float32)
  h = jnp.maximum(h, 0.0)
  scores = jnp.dot(h, a_ref[...], preferred_element_type=jnp.float32)
  scores += c_ref[rel:rel + 1, :]
  w = 1.0 / (1.0 + jnp.exp(-scores))
  hh = h.reshape(bm, H, DH)
  out_ref[...] = (hh * w[:, :, None]).reshape(bm, F)


def _tc_layer(dif_mat, rows, w_mat, a_mat, c_vec, rel, src_off, dst_off,
              bm=512):
  m, kk = dif_mat.shape
  grid = (m // bm,)
  so, do = src_off // kk, dst_off // bm
  return pl.pallas_call(
      functools.partial(_layer_body, bm=bm, rel=rel),
      grid=grid,
      in_specs=[
          pl.BlockSpec((bm, kk), lambda i: (i, 0)),
          pl.BlockSpec((kk, F), lambda i, so=so: (so, 0)),
          pl.BlockSpec((bm, F), lambda i, do=do: (do + i, 0)),
          pl.BlockSpec((F, F), lambda i: (0, 0)),
          pl.BlockSpec((F, F), lambda i: (1, 0)),
          pl.BlockSpec((F, H), lambda i: (0, 0)),
          pl.BlockSpec((2, H), lambda i: (0, 0)),
      ],
      out_specs=pl.BlockSpec((bm, F), lambda i: (i, 0)),
      out_shape=jax.ShapeDtypeStruct((m, F), jnp.float32),
  )(dif_mat, rows, rows, w_mat, w_mat, a_mat, c_vec)


def _final_body(x0_ref, x1_ref, wc_ref, out_ref):
  s = x0_ref[...] + x1_ref[...]
  n = s * lax.rsqrt(jnp.maximum(jnp.sum(s * s, axis=1, keepdims=True), 1e-12))
  logits = jnp.dot(n, wc_ref[...], preferred_element_type=jnp.float32)
  m = jnp.max(logits, axis=1, keepdims=True)
  e = jnp.exp(logits - m)
  out_ref[...] = e / jnp.sum(e, axis=1, keepdims=True)


def _tc_final(x2_0, x2_1, wc):
  n_cls = wc.shape[1]
  return pl.pallas_call(
      _final_body,
      out_shape=jax.ShapeDtypeStruct((N2, n_cls), jnp.float32),
  )(x2_0, x2_1, wc)


def kernel(features, src_nodes_0, dstsrc2src_0_1, dstsrc2dst_0_1, dif_mat_0_1,
           dstsrc2src_0_2, dstsrc2dst_0_2, dif_mat_0_2, src_nodes_1,
           dstsrc2src_1_1, dstsrc2dst_1_1, dif_mat_1_1, dstsrc2src_1_2,
           dstsrc2dst_1_2, dif_mat_1_2, W1, W2, attention_vec,
           relation_vectors, Wc):
  a1 = attention_vec[:F, 0]
  a2 = attention_vec[F:, 0].reshape(H, DH)
  heads = jnp.arange(F, dtype=jnp.int32) // DH
  a_mat = a1[:, None] * (heads[:, None] == jnp.arange(H)[None, :])
  c_vec = jnp.sum(relation_vectors.reshape(2, H, DH) * a2[None], axis=2)

  idx1 = jnp.concatenate([dstsrc2src_0_1, dstsrc2src_1_1 + N0,
                          dstsrc2dst_0_1,
                          dstsrc2dst_1_1 + N0]).reshape(32, 6, 128)
  sn_cat = jnp.concatenate([src_nodes_0, src_nodes_1])
  rows1 = _sc_gather_l1(features, sn_cat, idx1)

  x1_0 = _tc_layer(dif_mat_0_1, rows1, W1, a_mat, c_vec, 0, 0, 16384)
  x1_1 = _tc_layer(dif_mat_1_1, rows1, W1, a_mat, c_vec, 1, 8192, 20480)

  idx2 = jnp.concatenate([dstsrc2src_0_2, dstsrc2src_1_2 + N1,
                          dstsrc2dst_0_2,
                          dstsrc2dst_1_2 + N1]).reshape(32, 5, 64)
  rows2 = _sc_gather_l2(jnp.concatenate([x1_0, x1_1]), idx2)

  x2_0 = _tc_layer(dif_mat_0_2, rows2, W2, a_mat, c_vec, 0, 0, 8192)
  x2_1 = _tc_layer(dif_mat_1_2, rows2, W2, a_mat, c_vec, 1, 4096, 9216)

  return _tc_final(x2_0, x2_1, Wc)
